# baseline (speedup 1.0000x reference)
import functools

import jax
import jax.numpy as jnp
from jax import lax
from jax.experimental import pallas as pl
from jax.experimental.pallas import tpu as pltpu
from jax.experimental.pallas import tpu_sc as plsc

D = 768
N_PATCHES = 576
N_TOT = N_PATCHES + 1
BATCH = 128

NC = 2
NS = 16
NW = NC * NS
BPW = BATCH // NW
RC = 8
NCHUNK = N_PATCHES // RC
CHW = RC * D
LANES = 16
NSLOT = 8
CPG = NSLOT // BPW
NGROUP = NCHUNK // CPG
UNROLL = 8


def _sc_body(in_hbm, cls_hbm, pos_hbm, out_hbm,
             inbuf, outbuf, posbuf, clsbuf, *sems):
  in_sems = sems[:NSLOT]
  out_sems = sems[NSLOT:2 * NSLOT]
  pos_sems = sems[2 * NSLOT:]
  wid = lax.axis_index("c") * NS + lax.axis_index("s")
  b0 = BPW * wid

  def in_copy(c, i, s):
    off = pl.multiple_of(((b0 + i) * N_PATCHES + RC * c) * D, 8)
    return pltpu.make_async_copy(
        in_hbm.at[pl.ds(off, CHW)], inbuf.at[s], in_sems[s])

  def out_copy(c, i, s):
    off = pl.multiple_of(((b0 + i) * N_TOT + RC * c + 1) * D, 8)
    return pltpu.make_async_copy(
        outbuf.at[s], out_hbm.at[pl.ds(off, CHW)], out_sems[s])

  def pos_copy(c, ps):
    off = pl.multiple_of((RC * c + 1) * D, 8)
    return pltpu.make_async_copy(
        pos_hbm.at[pl.ds(off, CHW)], posbuf.at[ps], pos_sems[ps])

  pltpu.sync_copy(cls_hbm, clsbuf)
  pltpu.sync_copy(pos_hbm.at[pl.ds(0, D)], outbuf.at[0, pl.ds(0, D)])
  for k in range(D // LANES):
    sl = pl.ds(k * LANES, LANES)
    clsbuf[sl] = clsbuf[sl] + outbuf[0, sl]
  for i in range(BPW):
    off = pl.multiple_of((b0 + i) * N_TOT * D, 8)
    pltpu.sync_copy(clsbuf, out_hbm.at[pl.ds(off, D)])

  pos_copy(0, 0).start()
  pos_copy(1, 1).start()
  for t in range(NSLOT):
    in_copy(t // BPW, t % BPW, t).start()

  def group(g, carry):
    cbase = CPG * g
    for t in range(NSLOT):
      j = t // BPW
      i = t % BPW
      s = t
      c = cbase + j
      if i == 0:
        pos_copy(c, j % 2).wait()
      in_copy(c, i, s).wait()

      @pl.when(g > 0)
      def _():
        out_copy(c - CPG, i, s).wait()

      @plsc.parallel_loop(0, CHW, LANES, unroll=UNROLL)
      def _(off):
        sl = pl.ds(off, LANES)
        outbuf[s, sl] = inbuf[s, sl] + posbuf[j % 2, sl]

      out_copy(c, i, s).start()

      @pl.when(g < NGROUP - 1)
      def _():
        in_copy(c + CPG, i, s).start()

      if i == BPW - 1:
        @pl.when(g < NGROUP - 1)
        def _():
          pos_copy(c + CPG, j % 2).start()

    return carry

  lax.fori_loop(0, NGROUP, group, 0)
  for t in range(NSLOT):
    out_copy(CPG * (NGROUP - 1) + t // BPW, t % BPW, t).wait()


@jax.jit
def kernel(inputs, class_embed, pos_table):
  mesh = plsc.VectorSubcoreMesh(core_axis_name="c", subcore_axis_name="s")
  run = functools.partial(
      pl.kernel,
      mesh=mesh,
      out_type=jax.ShapeDtypeStruct((BATCH * N_TOT * D,), jnp.float32),
      scratch_types=(
          [
              pltpu.VMEM((NSLOT, CHW), jnp.float32),
              pltpu.VMEM((NSLOT, CHW), jnp.float32),
              pltpu.VMEM((2, CHW), jnp.float32),
              pltpu.VMEM((D,), jnp.float32),
          ]
          + [pltpu.SemaphoreType.DMA] * (2 * NSLOT + 2)
      ),
  )(_sc_body)
  out = run(inputs.reshape(-1), class_embed.reshape(-1), pos_table.reshape(-1))
  return out.reshape(BATCH, N_TOT, D)


# speedup vs baseline: 2.6442x; 2.6442x over previous
import functools

import jax
import jax.numpy as jnp
from jax import lax
from jax.experimental import pallas as pl
from jax.experimental.pallas import tpu as pltpu
from jax.experimental.pallas import tpu_sc as plsc

D = 768
N_PATCHES = 576
N_TOT = N_PATCHES + 1
BATCH = 128

NC = 2
NS = 16
NW = NC * NS
BPW = BATCH // NW
LANES = 16


def _sc_cls_body(cls_hbm, pos_hbm, out_hbm, clsbuf, posbuf):
  wid = lax.axis_index("c") * NS + lax.axis_index("s")
  b0 = BPW * wid
  pltpu.sync_copy(cls_hbm, clsbuf)
  pltpu.sync_copy(pos_hbm.at[pl.ds(0, D)], posbuf)
  for k in range(D // LANES):
    sl = pl.ds(k * LANES, LANES)
    clsbuf[sl] = clsbuf[sl] + posbuf[sl]
  for i in range(BPW):
    off = pl.multiple_of((b0 + i) * D, 8)
    pltpu.sync_copy(clsbuf, out_hbm.at[pl.ds(off, D)])


def _tc_body(cls_ref, x_ref, pos_ref, o_ref):
  o_ref[0, 0:1, :] = cls_ref[0]
  o_ref[0, 1:, :] = x_ref[0] + pos_ref[1:, :]


@jax.jit
def kernel(inputs, class_embed, pos_table):
  mesh = plsc.VectorSubcoreMesh(core_axis_name="c", subcore_axis_name="s")
  sc_cls = functools.partial(
      pl.kernel,
      mesh=mesh,
      out_type=jax.ShapeDtypeStruct((BATCH * D,), jnp.float32),
      scratch_types=[
          pltpu.VMEM((D,), jnp.float32),
          pltpu.VMEM((D,), jnp.float32),
      ],
  )(_sc_cls_body)
  cls_rows = sc_cls(class_embed.reshape(-1), pos_table.reshape(-1))
  cls_rows = cls_rows.reshape(BATCH, 1, D)

  out = pl.pallas_call(
      _tc_body,
      grid=(BATCH,),
      in_specs=[
          pl.BlockSpec((1, 1, D), lambda b: (b, 0, 0)),
          pl.BlockSpec((1, N_PATCHES, D), lambda b: (b, 0, 0)),
          pl.BlockSpec((N_TOT, D), lambda b: (0, 0)),
      ],
      out_specs=pl.BlockSpec((1, N_TOT, D), lambda b: (b, 0, 0)),
      out_shape=jax.ShapeDtypeStruct((BATCH, N_TOT, D), jnp.float32),
  )(cls_rows, inputs, pos_table)
  return out


# speedup vs baseline: 2.8916x; 1.0936x over previous
import functools

import jax
import jax.numpy as jnp
from jax import lax
from jax.experimental import pallas as pl
from jax.experimental.pallas import tpu as pltpu
from jax.experimental.pallas import tpu_sc as plsc

D = 768
N_PATCHES = 576
N_TOT = N_PATCHES + 1
BATCH = 128

NC = 2
NS = 16
NW = NC * NS
BPW = BATCH // NW
LANES = 16


def _sc_cls_body(cls_hbm, pos_hbm, out_hbm, clsbuf, posbuf):
  wid = lax.axis_index("c") * NS + lax.axis_index("s")
  b0 = BPW * wid
  pltpu.sync_copy(cls_hbm, clsbuf)
  pltpu.sync_copy(pos_hbm.at[pl.ds(0, D)], posbuf)
  for k in range(D // LANES):
    sl = pl.ds(k * LANES, LANES)
    clsbuf[sl] = clsbuf[sl] + posbuf[sl]
  for i in range(BPW):
    off = pl.multiple_of((b0 + i) * D, 8)
    pltpu.sync_copy(clsbuf, out_hbm.at[pl.ds(off, D)])


TCB = 4


def _tc_body(cls_ref, x_ref, pos_ref, o_ref):
  o_ref[:, 0:1, :] = cls_ref[...]
  o_ref[:, 1:, :] = x_ref[...] + pos_ref[1:, :]


@jax.jit
def kernel(inputs, class_embed, pos_table):
  mesh = plsc.VectorSubcoreMesh(core_axis_name="c", subcore_axis_name="s")
  sc_cls = functools.partial(
      pl.kernel,
      mesh=mesh,
      out_type=jax.ShapeDtypeStruct((BATCH * D,), jnp.float32),
      scratch_types=[
          pltpu.VMEM((D,), jnp.float32),
          pltpu.VMEM((D,), jnp.float32),
      ],
  )(_sc_cls_body)
  cls_rows = sc_cls(class_embed.reshape(-1), pos_table.reshape(-1))
  cls_rows = cls_rows.reshape(BATCH, 1, D)

  out = pl.pallas_call(
      _tc_body,
      grid=(BATCH // TCB,),
      in_specs=[
          pl.BlockSpec((TCB, 1, D), lambda b: (b, 0, 0)),
          pl.BlockSpec((TCB, N_PATCHES, D), lambda b: (b, 0, 0)),
          pl.BlockSpec((N_TOT, D), lambda b: (0, 0)),
      ],
      out_specs=pl.BlockSpec((TCB, N_TOT, D), lambda b: (b, 0, 0)),
      out_shape=jax.ShapeDtypeStruct((BATCH, N_TOT, D), jnp.float32),
  )(cls_rows, inputs, pos_table)
  return out


# speedup vs baseline: 2.9075x; 1.0055x over previous
import functools

import jax
import jax.numpy as jnp
from jax import lax
from jax.experimental import pallas as pl
from jax.experimental.pallas import tpu as pltpu
from jax.experimental.pallas import tpu_sc as plsc

D = 768
N_PATCHES = 576
N_TOT = N_PATCHES + 1
BATCH = 128

NC = 2
NS = 16
NW = NC * NS
BPW = BATCH // NW
LANES = 16


def _sc_cls_body(cls_hbm, pos_hbm, out_hbm, clsbuf, posbuf):
  wid = lax.axis_index("c") * NS + lax.axis_index("s")
  b0 = BPW * wid
  pltpu.sync_copy(cls_hbm, clsbuf)
  pltpu.sync_copy(pos_hbm.at[pl.ds(0, D)], posbuf)
  for k in range(D // LANES):
    sl = pl.ds(k * LANES, LANES)
    clsbuf[sl] = clsbuf[sl] + posbuf[sl]
  for i in range(BPW):
    off = pl.multiple_of((b0 + i) * D, 8)
    pltpu.sync_copy(clsbuf, out_hbm.at[pl.ds(off, D)])


TCB = 8


def _tc_body(cls_ref, x_ref, pos_ref, o_ref):
  o_ref[:, 0:1, :] = cls_ref[...]
  o_ref[:, 1:, :] = x_ref[...] + pos_ref[1:, :]


@jax.jit
def kernel(inputs, class_embed, pos_table):
  mesh = plsc.VectorSubcoreMesh(core_axis_name="c", subcore_axis_name="s")
  sc_cls = functools.partial(
      pl.kernel,
      mesh=mesh,
      out_type=jax.ShapeDtypeStruct((BATCH * D,), jnp.float32),
      scratch_types=[
          pltpu.VMEM((D,), jnp.float32),
          pltpu.VMEM((D,), jnp.float32),
      ],
  )(_sc_cls_body)
  cls_rows = sc_cls(class_embed.reshape(-1), pos_table.reshape(-1))
  cls_rows = cls_rows.reshape(BATCH, 1, D)

  out = pl.pallas_call(
      _tc_body,
      grid=(BATCH // TCB,),
      in_specs=[
          pl.BlockSpec((TCB, 1, D), lambda b: (b, 0, 0)),
          pl.BlockSpec((TCB, N_PATCHES, D), lambda b: (b, 0, 0)),
          pl.BlockSpec((N_TOT, D), lambda b: (0, 0)),
      ],
      out_specs=pl.BlockSpec((TCB, N_TOT, D), lambda b: (b, 0, 0)),
      out_shape=jax.ShapeDtypeStruct((BATCH, N_TOT, D), jnp.float32),
  )(cls_rows, inputs, pos_table)
  return out


# speedup vs baseline: 2.9244x; 1.0058x over previous
import functools

import jax
import jax.numpy as jnp
from jax import lax
from jax.experimental import pallas as pl
from jax.experimental.pallas import tpu as pltpu
from jax.experimental.pallas import tpu_sc as plsc

D = 768
N_PATCHES = 576
N_TOT = N_PATCHES + 1
BATCH = 128

NC = 2
NS = 16
NW = NC * NS
BPW = BATCH // NW
LANES = 16


def _sc_cls_body(cls_hbm, pos_hbm, out_hbm, clsbuf, posbuf):
  wid = lax.axis_index("c") * NS + lax.axis_index("s")

  @pl.when(wid == 0)
  def _():
    pltpu.sync_copy(cls_hbm, clsbuf)
    pltpu.sync_copy(pos_hbm.at[pl.ds(0, D)], posbuf)
    for k in range(D // LANES):
      sl = pl.ds(k * LANES, LANES)
      clsbuf[sl] = clsbuf[sl] + posbuf[sl]
    pltpu.sync_copy(clsbuf, out_hbm)


TCB = 8


def _tc_body(cls_ref, x_ref, pos_ref, o_ref):
  o_ref[:, 0:1, :] = jnp.broadcast_to(cls_ref[...], (TCB, 1, D))
  o_ref[:, 1:, :] = x_ref[...] + pos_ref[1:, :]


@jax.jit
def kernel(inputs, class_embed, pos_table):
  mesh = plsc.VectorSubcoreMesh(core_axis_name="c", subcore_axis_name="s")
  sc_cls = functools.partial(
      pl.kernel,
      mesh=mesh,
      out_type=jax.ShapeDtypeStruct((D,), jnp.float32),
      scratch_types=[
          pltpu.VMEM((D,), jnp.float32),
          pltpu.VMEM((D,), jnp.float32),
      ],
  )(_sc_cls_body)
  cls_row = sc_cls(class_embed.reshape(-1), pos_table.reshape(-1))
  cls_row = cls_row.reshape(1, D)

  out = pl.pallas_call(
      _tc_body,
      grid=(BATCH // TCB,),
      in_specs=[
          pl.BlockSpec((1, D), lambda b: (0, 0)),
          pl.BlockSpec((TCB, N_PATCHES, D), lambda b: (b, 0, 0)),
          pl.BlockSpec((N_TOT, D), lambda b: (0, 0)),
      ],
      out_specs=pl.BlockSpec((TCB, N_TOT, D), lambda b: (b, 0, 0)),
      out_shape=jax.ShapeDtypeStruct((BATCH, N_TOT, D), jnp.float32),
  )(cls_row, inputs, pos_table)
  return out
